# baseline (speedup 1.0000x reference)
import functools

import jax
import jax.numpy as jnp
from jax import lax
from jax.experimental import pallas as pl
from jax.experimental.pallas import tpu as pltpu
from jax.experimental.pallas import tpu_sc as plsc

R = 128
N = 32768
K = 64
L = 16
NW = 32
ROWS_PER_W = R // NW
NCHUNK = N // L
NBINS = 256
HIST_WORDS = NBINS * L
MININT = -(1 << 31)
CAND_KEY_MIN = 0x40000000


def _sc_row_kth(x):
    mesh = plsc.VectorSubcoreMesh(core_axis_name="c", subcore_axis_name="s")

    @functools.partial(
        pl.kernel,
        mesh=mesh,
        out_type=jax.ShapeDtypeStruct((NW, L), jnp.int32),
        compiler_params=pltpu.CompilerParams(needs_layout_passes=False),
        scratch_types=[
            pltpu.VMEM((N,), jnp.float32),
            pltpu.VMEM((N,), jnp.float32),
            pltpu.VMEM((N,), jnp.int32),
            pltpu.VMEM((HIST_WORDS,), jnp.int32),
            pltpu.VMEM((L,), jnp.int32),
            pltpu.SemaphoreType.DMA,
            pltpu.SemaphoreType.DMA,
        ],
    )
    def k(x_hbm, out_hbm, buf0, buf1, cand, hist, stage, sem0, sem1):
        w = lax.axis_index("s") * 2 + lax.axis_index("c")
        row0 = w * ROWS_PER_W
        iota = lax.iota(jnp.int32, L)
        ones = jnp.ones((L,), jnp.int32)
        zeros = jnp.zeros((L,), jnp.int32)

        def zero_hist():
            @plsc.parallel_loop(0, NBINS, unroll=16)
            def _(i):
                hist[pl.ds(i * L, L)] = zeros

        def merge_group(j):
            h = zeros
            base = j * (L * L)
            for t in range(L):
                h = h + plsc.load_gather(hist, [base + iota * L + t])
            return h

        def scan_hist(kk):
            def bodyA(jj, st):
                enc_best, carry = st
                j = 15 - jj
                h = merge_group(j)
                cs = plsc.cumsum(h)
                tot = jnp.max(cs)
                cnt = (carry + tot) - cs + h
                above = jnp.minimum(cnt - h, 255)
                bvec = j * L + iota
                enc = jnp.where(
                    cnt >= kk, jnp.left_shift(bvec, 8) + (255 - above), -1)
                return jnp.maximum(enc_best, jnp.max(enc)), carry + tot

            enc, _ = lax.fori_loop(
                0, 16, bodyA, (jnp.int32(-1), jnp.int32(0)))
            return jnp.right_shift(enc, 8), 255 - (enc & 255)

        def transform(v):
            u = plsc.bitcast(v, jnp.int32)
            sgn = jnp.right_shift(u, 31)
            key_s = u ^ (sgn & 0x7FFFFFFF)
            return key_s, key_s ^ MININT

        def process_row(buf):
            @plsc.parallel_loop(0, NCHUNK, unroll=16, carry=zeros)
            def offv(i, off):
                key_s, kb = transform(buf[pl.ds(i * L, L)])
                cm = key_s >= CAND_KEY_MIN
                plsc.store_scatter(cand, [off * L + iota], kb, mask=cm)
                return off + jnp.where(cm, 1, 0)

            m = jnp.sum(offv)

            def fb():
                @plsc.parallel_loop(0, NCHUNK, unroll=8)
                def _(i):
                    _, kb = transform(buf[pl.ds(i * L, L)])
                    cand[pl.ds(i * L, L)] = kb
                return jnp.full((L,), NCHUNK, jnp.int32)

            offv = lax.cond(m >= K, lambda: offv, fb)
            pmax = jnp.max(offv)

            kk = jnp.int32(K)
            prefix = jnp.int32(0)
            for shift in (24, 16, 8, 0):
                zero_hist()
                himask = jnp.int32(-(1 << (shift + 8))) if shift < 24 else 0

                @plsc.parallel_loop(0, pmax, unroll=4)
                def _(p, himask=himask, shift=shift, prefix=prefix,
                      offv=offv):
                    kv = cand[pl.ds(p * L, L)]
                    valid = (p < offv) & ((kv & himask) == prefix)
                    b = lax.shift_right_logical(kv, shift) & 0xFF
                    plsc.addupdate_scatter(
                        hist, [b * L + iota], ones, mask=valid)

                bl, sl = scan_hist(kk)
                prefix = prefix | jnp.left_shift(bl, shift)
                kk = kk - sl

            return jnp.broadcast_to(prefix ^ MININT, (L,))

        bufs = (buf0, buf1)
        sems = (sem0, sem1)
        cps = [None] * ROWS_PER_W
        cps[0] = pltpu.make_async_copy(x_hbm.at[row0], buf0, sem0)
        cps[0].start()
        acc = jnp.full((L,), 0x7FFFFFFF, jnp.int32)
        for r in range(ROWS_PER_W):
            if r + 1 < ROWS_PER_W:
                cps[r + 1] = pltpu.make_async_copy(
                    x_hbm.at[row0 + (r + 1)], bufs[(r + 1) % 2],
                    sems[(r + 1) % 2])
                cps[r + 1].start()
            cps[r].wait()
            acc = jnp.minimum(acc, process_row(bufs[r % 2]))

        stage[...] = acc
        pltpu.sync_copy(stage, out_hbm.at[w])

    return k(x)


def _tc_mask(x, kv8):
    def body(kv_ref, x_ref, o_ref):
        kmin = jnp.min(kv_ref[...])
        sgn = jnp.right_shift(kmin, 31)
        t = lax.bitcast_convert_type(kmin ^ (sgn & 0x7FFFFFFF), jnp.float32)
        xv = x_ref[...]
        o_ref[...] = jnp.where(xv >= t, xv, 0.0)

    grid = (8,)
    return pl.pallas_call(
        body,
        grid=grid,
        in_specs=[
            pl.BlockSpec((4, 128), lambda i: (0, 0)),
            pl.BlockSpec((16, N), lambda i: (i, 0)),
        ],
        out_specs=pl.BlockSpec((16, N), lambda i: (i, 0)),
        out_shape=jax.ShapeDtypeStruct((R, N), jnp.float32),
    )(kv8, x)


def kernel(inputs):
    kv = _sc_row_kth(inputs)
    kv8 = kv.reshape(4, 128)
    return _tc_mask(inputs, kv8)


# speedup vs baseline: 1.1147x; 1.1147x over previous
import functools

import jax
import jax.numpy as jnp
from jax import lax
from jax.experimental import pallas as pl
from jax.experimental.pallas import tpu as pltpu
from jax.experimental.pallas import tpu_sc as plsc

R = 128
N = 32768
K = 64
L = 16
NW = 32
ROWS_PER_W = R // NW
NCHUNK = N // L
NBINS = 256
HIST_WORDS = NBINS * L
MININT = -(1 << 31)
CAND_KEY_MIN = 0x40000000


def _sc_row_kth(x):
    mesh = plsc.VectorSubcoreMesh(core_axis_name="c", subcore_axis_name="s")

    @functools.partial(
        pl.kernel,
        mesh=mesh,
        out_type=jax.ShapeDtypeStruct((NW, L), jnp.int32),
        compiler_params=pltpu.CompilerParams(needs_layout_passes=False),
        scratch_types=[
            pltpu.VMEM((N,), jnp.float32),
            pltpu.VMEM((N,), jnp.float32),
            pltpu.VMEM((N,), jnp.int32),
            pltpu.VMEM((HIST_WORDS,), jnp.int32),
            pltpu.VMEM((L * L,), jnp.int32),
            pltpu.VMEM((L,), jnp.int32),
            pltpu.SemaphoreType.DMA,
            pltpu.SemaphoreType.DMA,
        ],
    )
    def k(x_hbm, out_hbm, buf0, buf1, cand, hist, chist, stage, sem0, sem1):
        w = lax.axis_index("s") * 2 + lax.axis_index("c")
        row0 = w * ROWS_PER_W
        iota = lax.iota(jnp.int32, L)
        ones = jnp.ones((L,), jnp.int32)
        zeros = jnp.zeros((L,), jnp.int32)

        def zero_hist():
            @plsc.parallel_loop(0, NBINS, unroll=16)
            def _(i):
                hist[pl.ds(i * L, L)] = zeros

        def merge_group(j):
            h = zeros
            base = j * (L * L)
            for t in range(L):
                h = h + plsc.load_gather(hist, [base + iota * L + t])
            return h

        def scan_hist(kk):
            mg = zeros
            for t in range(L):
                mg = mg + plsc.load_gather(chist, [iota * L + t])
            css = plsc.cumsum(mg)
            tot = jnp.max(css)
            cntg = tot - css + mg
            aboveg = jnp.minimum(cntg - mg, 65535)
            encg = jnp.where(
                cntg >= kk, jnp.left_shift(iota, 16) + (65535 - aboveg), -1)
            eg = jnp.max(encg)
            jg = jnp.right_shift(eg, 16)
            sg = 65535 - (eg & 65535)

            h = merge_group(jg)
            cs = plsc.cumsum(h)
            totg = jnp.max(cs)
            cnt = sg + (totg - cs) + h
            above = jnp.minimum(cnt - h, 255)
            bvec = jg * L + iota
            enc = jnp.where(
                cnt >= kk, jnp.left_shift(bvec, 8) + (255 - above), -1)
            e = jnp.max(enc)
            return jnp.right_shift(e, 8), 255 - (e & 255)

        def transform(v):
            u = plsc.bitcast(v, jnp.int32)
            sgn = jnp.right_shift(u, 31)
            key_s = u ^ (sgn & 0x7FFFFFFF)
            return key_s, key_s ^ MININT

        def process_row(buf):
            @plsc.parallel_loop(0, NCHUNK, unroll=16, carry=zeros)
            def offv(i, off):
                key_s, kb = transform(buf[pl.ds(i * L, L)])
                cm = key_s >= CAND_KEY_MIN
                plsc.store_scatter(cand, [off * L + iota], kb, mask=cm)
                return off + jnp.where(cm, 1, 0)

            m = jnp.sum(offv)

            def fb():
                @plsc.parallel_loop(0, NCHUNK, unroll=8)
                def _(i):
                    _, kb = transform(buf[pl.ds(i * L, L)])
                    cand[pl.ds(i * L, L)] = kb
                return jnp.full((L,), NCHUNK, jnp.int32)

            offv = lax.cond(m >= K, lambda: offv, fb)
            pmax = jnp.max(offv)

            kk = jnp.int32(K)
            prefix = jnp.int32(0)
            for shift in (24, 16, 8, 0):
                zero_hist()
                for t in range(L):
                    chist[pl.ds(t * L, L)] = zeros

                @plsc.parallel_loop(0, pmax, unroll=4)
                def _(p, himask=(jnp.int32(-(1 << (shift + 8)))
                                 if shift < 24 else 0),
                      shift=shift, prefix=prefix, offv=offv):
                    kv = cand[pl.ds(p * L, L)]
                    valid = (p < offv) & ((kv & himask) == prefix)
                    b = lax.shift_right_logical(kv, shift) & 0xFF
                    plsc.addupdate_scatter(
                        hist, [b * L + iota], ones, mask=valid)
                    cb = lax.shift_right_logical(kv, shift + 4) & 0xF
                    plsc.addupdate_scatter(
                        chist, [cb * L + iota], ones, mask=valid)

                bl, sl = scan_hist(kk)
                prefix = prefix | jnp.left_shift(bl, shift)
                kk = kk - sl

            return jnp.broadcast_to(prefix ^ MININT, (L,))

        bufs = (buf0, buf1)
        sems = (sem0, sem1)
        cps = [None] * ROWS_PER_W
        cps[0] = pltpu.make_async_copy(x_hbm.at[row0], buf0, sem0)
        cps[0].start()
        acc = jnp.full((L,), 0x7FFFFFFF, jnp.int32)
        for r in range(ROWS_PER_W):
            if r + 1 < ROWS_PER_W:
                cps[r + 1] = pltpu.make_async_copy(
                    x_hbm.at[row0 + (r + 1)], bufs[(r + 1) % 2],
                    sems[(r + 1) % 2])
                cps[r + 1].start()
            cps[r].wait()
            acc = jnp.minimum(acc, process_row(bufs[r % 2]))

        stage[...] = acc
        pltpu.sync_copy(stage, out_hbm.at[w])

    return k(x)


def _tc_mask(x, kv8):
    def body(kv_ref, x_ref, o_ref):
        kmin = jnp.min(kv_ref[...])
        sgn = jnp.right_shift(kmin, 31)
        t = lax.bitcast_convert_type(kmin ^ (sgn & 0x7FFFFFFF), jnp.float32)
        xv = x_ref[...]
        o_ref[...] = jnp.where(xv >= t, xv, 0.0)

    grid = (4,)
    return pl.pallas_call(
        body,
        grid=grid,
        in_specs=[
            pl.BlockSpec((4, 128), lambda i: (0, 0)),
            pl.BlockSpec((32, N), lambda i: (i, 0)),
        ],
        out_specs=pl.BlockSpec((32, N), lambda i: (i, 0)),
        out_shape=jax.ShapeDtypeStruct((R, N), jnp.float32),
    )(kv8, x)


def kernel(inputs):
    kv = _sc_row_kth(inputs)
    kv8 = kv.reshape(4, 128)
    return _tc_mask(inputs, kv8)


# speedup vs baseline: 1.1446x; 1.0268x over previous
import functools

import jax
import jax.numpy as jnp
from jax import lax
from jax.experimental import pallas as pl
from jax.experimental.pallas import tpu as pltpu
from jax.experimental.pallas import tpu_sc as plsc

R = 128
N = 32768
K = 64
L = 16
NW = 32
ROWS_PER_W = R // NW
NCHUNK = N // L
NBINS = 256
HIST_WORDS = NBINS * L
MININT = -(1 << 31)
CAND_KEY_MIN = 0x40000000


def _sc_row_kth(x):
    mesh = plsc.VectorSubcoreMesh(core_axis_name="c", subcore_axis_name="s")

    @functools.partial(
        pl.kernel,
        mesh=mesh,
        out_type=jax.ShapeDtypeStruct((NW, L), jnp.int32),
        compiler_params=pltpu.CompilerParams(needs_layout_passes=False),
        scratch_types=[
            pltpu.VMEM((N,), jnp.float32),
            pltpu.VMEM((N,), jnp.float32),
            pltpu.VMEM((N,), jnp.int32),
            pltpu.VMEM((HIST_WORDS,), jnp.int32),
            pltpu.VMEM((L * L,), jnp.int32),
            pltpu.VMEM((L,), jnp.int32),
            pltpu.SemaphoreType.DMA,
            pltpu.SemaphoreType.DMA,
        ],
    )
    def k(x_hbm, out_hbm, buf0, buf1, cand, hist, chist, stage, sem0, sem1):
        w = lax.axis_index("s") * 2 + lax.axis_index("c")
        row0 = w * ROWS_PER_W
        iota = lax.iota(jnp.int32, L)
        ones = jnp.ones((L,), jnp.int32)
        neg_ones = jnp.full((L,), -1, jnp.int32)
        zeros = jnp.zeros((L,), jnp.int32)

        def zero_hist():
            @plsc.parallel_loop(0, NBINS, unroll=16)
            def _(i):
                hist[pl.ds(i * L, L)] = zeros

        def merge_group(j):
            h = zeros
            base = j * (L * L)
            for t in range(L):
                h = h + plsc.load_gather(hist, [base + iota * L + t])
            return h

        def scan_hist(kk):
            mg = zeros
            for t in range(L):
                mg = mg + plsc.load_gather(chist, [iota * L + t])
            css = plsc.cumsum(mg)
            tot = jnp.max(css)
            cntg = tot - css + mg
            aboveg = jnp.minimum(cntg - mg, 65535)
            encg = jnp.where(
                cntg >= kk, jnp.left_shift(iota, 16) + (65535 - aboveg), -1)
            eg = jnp.max(encg)
            jg = jnp.right_shift(eg, 16)
            sg = 65535 - (eg & 65535)

            h = merge_group(jg)
            cs = plsc.cumsum(h)
            totg = jnp.max(cs)
            cnt = sg + (totg - cs) + h
            above = jnp.minimum(cnt - h, 255)
            bvec = jg * L + iota
            enc = jnp.where(
                cnt >= kk, jnp.left_shift(bvec, 8) + (255 - above), -1)
            e = jnp.max(enc)
            return jnp.right_shift(e, 8), 255 - (e & 255)

        def transform(v):
            u = plsc.bitcast(v, jnp.int32)
            sgn = jnp.right_shift(u, 31)
            return u, u ^ (sgn | MININT)

        def level_pass(shift, prefix, offv, pmax, delta):
            himask = jnp.int32(-(1 << (shift + 8))) if shift < 24 else 0

            @plsc.parallel_loop(0, pmax, unroll=4)
            def _(p):
                kv = cand[pl.ds(p * L, L)]
                valid = (p < offv) & ((kv & himask) == prefix)
                b = lax.shift_right_logical(kv, shift) & 0xFF
                plsc.addupdate_scatter(
                    hist, [b * L + iota], delta, mask=valid)
                cb = lax.shift_right_logical(kv, shift + 4) & 0xF
                plsc.addupdate_scatter(
                    chist, [cb * L + iota], delta, mask=valid)

        def process_row(buf):
            @plsc.parallel_loop(0, NCHUNK, unroll=16, carry=zeros)
            def offv(i, off):
                u, kb = transform(buf[pl.ds(i * L, L)])
                cm = u >= CAND_KEY_MIN
                plsc.store_scatter(cand, [off * L + iota], kb, mask=cm)
                return off + jnp.where(cm, 1, 0)

            m = jnp.sum(offv)

            def fb():
                @plsc.parallel_loop(0, NCHUNK, unroll=8)
                def _(i):
                    _, kb = transform(buf[pl.ds(i * L, L)])
                    cand[pl.ds(i * L, L)] = kb
                return jnp.full((L,), NCHUNK, jnp.int32)

            offv = lax.cond(m >= K, lambda: offv, fb)
            pmax = jnp.max(offv)

            kk = jnp.int32(K)
            prefix = jnp.int32(0)
            for shift in (24, 16, 8, 0):
                level_pass(shift, prefix, offv, pmax, ones)
                bl, sl = scan_hist(kk)
                level_pass(shift, prefix, offv, pmax, neg_ones)
                prefix = prefix | jnp.left_shift(bl, shift)
                kk = kk - sl

            return jnp.broadcast_to(prefix ^ MININT, (L,))

        bufs = (buf0, buf1)
        sems = (sem0, sem1)
        cps = [None] * ROWS_PER_W
        cps[0] = pltpu.make_async_copy(x_hbm.at[row0], buf0, sem0)
        cps[0].start()
        zero_hist()
        for t in range(L):
            chist[pl.ds(t * L, L)] = zeros
        acc = jnp.full((L,), 0x7FFFFFFF, jnp.int32)
        for r in range(ROWS_PER_W):
            if r + 1 < ROWS_PER_W:
                cps[r + 1] = pltpu.make_async_copy(
                    x_hbm.at[row0 + (r + 1)], bufs[(r + 1) % 2],
                    sems[(r + 1) % 2])
                cps[r + 1].start()
            cps[r].wait()
            acc = jnp.minimum(acc, process_row(bufs[r % 2]))

        stage[...] = acc
        pltpu.sync_copy(stage, out_hbm.at[w])

    return k(x)


def _tc_mask(x, kv8):
    def body(kv_ref, x_ref, o_ref):
        kmin = jnp.min(kv_ref[...])
        sgn = jnp.right_shift(kmin, 31)
        t = lax.bitcast_convert_type(kmin ^ (sgn & 0x7FFFFFFF), jnp.float32)
        xv = x_ref[...]
        o_ref[...] = jnp.where(xv >= t, xv, 0.0)

    grid = (4,)
    return pl.pallas_call(
        body,
        grid=grid,
        in_specs=[
            pl.BlockSpec((4, 128), lambda i: (0, 0)),
            pl.BlockSpec((32, N), lambda i: (i, 0)),
        ],
        out_specs=pl.BlockSpec((32, N), lambda i: (i, 0)),
        out_shape=jax.ShapeDtypeStruct((R, N), jnp.float32),
    )(kv8, x)


def kernel(inputs):
    kv = _sc_row_kth(inputs)
    kv8 = kv.reshape(4, 128)
    return _tc_mask(inputs, kv8)


# speedup vs baseline: 1.1841x; 1.0346x over previous
import functools

import jax
import jax.numpy as jnp
from jax import lax
from jax.experimental import pallas as pl
from jax.experimental.pallas import tpu as pltpu
from jax.experimental.pallas import tpu_sc as plsc

R = 128
N = 32768
K = 64
L = 16
NW = 32
ROWS_PER_W = R // NW
NCHUNK = N // L
NBINS = 256
HIST_WORDS = NBINS * L
MININT = -(1 << 31)
CAND_KEY_MIN = 0x40200000


def _sc_row_kth(x):
    mesh = plsc.VectorSubcoreMesh(core_axis_name="c", subcore_axis_name="s")

    @functools.partial(
        pl.kernel,
        mesh=mesh,
        out_type=jax.ShapeDtypeStruct((NW, L), jnp.int32),
        compiler_params=pltpu.CompilerParams(needs_layout_passes=False),
        scratch_types=[
            pltpu.VMEM((N,), jnp.float32),
            pltpu.VMEM((N,), jnp.float32),
            pltpu.VMEM((N,), jnp.int32),
            pltpu.VMEM((HIST_WORDS,), jnp.int32),
            pltpu.VMEM((L * L,), jnp.int32),
            pltpu.VMEM((L,), jnp.int32),
            pltpu.SemaphoreType.DMA,
            pltpu.SemaphoreType.DMA,
        ],
    )
    def k(x_hbm, out_hbm, buf0, buf1, cand, hist, chist, stage, sem0, sem1):
        w = lax.axis_index("s") * 2 + lax.axis_index("c")
        row0 = w * ROWS_PER_W
        iota = lax.iota(jnp.int32, L)
        ones = jnp.ones((L,), jnp.int32)
        neg_ones = jnp.full((L,), -1, jnp.int32)
        zeros = jnp.zeros((L,), jnp.int32)

        def zero_hist():
            @plsc.parallel_loop(0, NBINS, unroll=16)
            def _(i):
                hist[pl.ds(i * L, L)] = zeros

        def merge_group(j):
            h = zeros
            base = j * (L * L)
            for t in range(L):
                h = h + plsc.load_gather(hist, [base + iota * L + t])
            return h

        def scan_hist(kk):
            mg = zeros
            for t in range(L):
                mg = mg + plsc.load_gather(chist, [iota * L + t])
            css = plsc.cumsum(mg)
            tot = jnp.max(css)
            cntg = tot - css + mg
            aboveg = jnp.minimum(cntg - mg, 65535)
            encg = jnp.where(
                cntg >= kk, jnp.left_shift(iota, 16) + (65535 - aboveg), -1)
            eg = jnp.max(encg)
            jg = jnp.right_shift(eg, 16)
            sg = 65535 - (eg & 65535)

            h = merge_group(jg)
            cs = plsc.cumsum(h)
            totg = jnp.max(cs)
            cnt = sg + (totg - cs) + h
            above = jnp.minimum(cnt - h, 255)
            bvec = jg * L + iota
            enc = jnp.where(
                cnt >= kk, jnp.left_shift(bvec, 8) + (255 - above), -1)
            e = jnp.max(enc)
            return jnp.right_shift(e, 8), 255 - (e & 255)

        def transform(v):
            u = plsc.bitcast(v, jnp.int32)
            sgn = jnp.right_shift(u, 31)
            return u, u ^ (sgn | MININT)

        def level_pass(shift, prefix, offv, pmax, delta):
            himask = jnp.int32(-(1 << (shift + 8))) if shift < 24 else 0

            @plsc.parallel_loop(0, pmax, unroll=4)
            def _(p):
                kv = cand[pl.ds(p * L, L)]
                valid = (p < offv) & ((kv & himask) == prefix)
                b = lax.shift_right_logical(kv, shift) & 0xFF
                plsc.addupdate_scatter(
                    hist, [b * L + iota], delta, mask=valid)
                cb = lax.shift_right_logical(kv, shift + 4) & 0xF
                plsc.addupdate_scatter(
                    chist, [cb * L + iota], delta, mask=valid)

        def process_row(buf):
            @plsc.parallel_loop(0, NCHUNK, unroll=16, carry=zeros)
            def offv(i, off):
                u, kb = transform(buf[pl.ds(i * L, L)])
                cm = u >= CAND_KEY_MIN
                plsc.store_scatter(cand, [off * L + iota], kb, mask=cm)
                return off + jnp.where(cm, 1, 0)

            m = jnp.sum(offv)

            def fb():
                @plsc.parallel_loop(0, NCHUNK, unroll=8)
                def _(i):
                    _, kb = transform(buf[pl.ds(i * L, L)])
                    cand[pl.ds(i * L, L)] = kb
                return jnp.full((L,), NCHUNK, jnp.int32)

            offv = lax.cond(m >= K, lambda: offv, fb)
            pmax = jnp.max(offv)

            kk = jnp.int32(K)
            prefix = jnp.int32(0)
            for shift in (24, 16, 8, 0):
                level_pass(shift, prefix, offv, pmax, ones)
                bl, sl = scan_hist(kk)
                level_pass(shift, prefix, offv, pmax, neg_ones)
                prefix = prefix | jnp.left_shift(bl, shift)
                kk = kk - sl

            return jnp.broadcast_to(prefix ^ MININT, (L,))

        bufs = (buf0, buf1)
        sems = (sem0, sem1)
        cps = [None] * ROWS_PER_W
        cps[0] = pltpu.make_async_copy(x_hbm.at[row0], buf0, sem0)
        cps[0].start()
        zero_hist()
        for t in range(L):
            chist[pl.ds(t * L, L)] = zeros
        acc = jnp.full((L,), 0x7FFFFFFF, jnp.int32)
        for r in range(ROWS_PER_W):
            if r + 1 < ROWS_PER_W:
                cps[r + 1] = pltpu.make_async_copy(
                    x_hbm.at[row0 + (r + 1)], bufs[(r + 1) % 2],
                    sems[(r + 1) % 2])
                cps[r + 1].start()
            cps[r].wait()
            acc = jnp.minimum(acc, process_row(bufs[r % 2]))

        stage[...] = acc
        pltpu.sync_copy(stage, out_hbm.at[w])

    return k(x)


def _tc_mask(x, kv8):
    def body(kv_ref, x_ref, o_ref):
        kmin = jnp.min(kv_ref[...])
        sgn = jnp.right_shift(kmin, 31)
        t = lax.bitcast_convert_type(kmin ^ (sgn & 0x7FFFFFFF), jnp.float32)
        xv = x_ref[...]
        o_ref[...] = jnp.where(xv >= t, xv, 0.0)

    grid = (4,)
    return pl.pallas_call(
        body,
        grid=grid,
        in_specs=[
            pl.BlockSpec((4, 128), lambda i: (0, 0)),
            pl.BlockSpec((32, N), lambda i: (i, 0)),
        ],
        out_specs=pl.BlockSpec((32, N), lambda i: (i, 0)),
        out_shape=jax.ShapeDtypeStruct((R, N), jnp.float32),
    )(kv8, x)


def kernel(inputs):
    kv = _sc_row_kth(inputs)
    kv8 = kv.reshape(4, 128)
    return _tc_mask(inputs, kv8)
